# baseline (speedup 1.0000x reference)
import dataclasses
import functools

import numpy as np
import jax
import jax.numpy as jnp
from jax import lax
from jax.experimental import pallas as pl
from jax.experimental.pallas import tpu as pltpu
from jax.experimental.pallas import tpu_sc as plsc

NF = 26
D = 16
P = NF * (NF - 1) // 2
NIDX = 2 * P
NIDX_PAD = 768
VAL_PAD = 32
NC, NS = 2, 16
NW = NC * NS

_IU, _JU = np.triu_indices(NF, k=1)


def _ffm_body(rows_per_w, emb_hbm, idx_hbm, val_hbm, out_hbm,
              idx_v, val_v, rows_v, acc_v, out_v, sem):
    core = lax.axis_index("c")
    sub = lax.axis_index("s")
    wid = sub * NC + core
    base = wid * rows_per_w
    acc_v[...] = jnp.zeros((D,), jnp.float32)

    @pl.loop(0, rows_per_w)
    def _row(r):
        row = base + r
        pltpu.sync_copy(idx_hbm.at[row], idx_v)
        pltpu.sync_copy(val_hbm.at[row], val_v)
        copies = []
        for c in range(6):
            copies.append(pltpu.async_copy(
                emb_hbm.at[idx_v.at[c]],
                rows_v.at[pl.ds(c * 128, 128)], sem))
        for cp in copies:
            cp.wait()

        vbc = [val_v[i] for i in range(NF)]
        acc = jnp.zeros((D,), jnp.float32)
        p = 0
        for i in range(NF - 1):
            acc_i = jnp.zeros((D,), jnp.float32)
            for j in range(i + 1, NF):
                acc_i = acc_i + rows_v[p] * rows_v[P + p] * vbc[j]
                p += 1
            acc = acc + acc_i * vbc[i]
        s = jnp.sum(acc)

        lane = lax.rem(r, D)
        sel = lax.iota(jnp.int32, D) == lane
        acc_v[...] = acc_v[...] + jnp.where(sel, s, jnp.float32(0.0))

        @pl.when(lane == D - 1)
        def _flush():
            off = pl.multiple_of((r // D) * D, D)
            out_v[pl.ds(off, D)] = acc_v[...]
            acc_v[...] = jnp.zeros((D,), jnp.float32)

    pltpu.sync_copy(out_v, out_hbm.at[pl.ds(base, rows_per_w)])


def kernel(x_field, x, x_val, emb):
    batch = x.shape[0]
    total = emb.shape[1]
    rows_per_w = batch // NW

    emb_flat = emb.reshape(NF * total, D)
    iu = jnp.asarray(_IU)
    ju = jnp.asarray(_JU)
    xf = x_field.astype(jnp.int32)
    xi = x.astype(jnp.int32)
    vi_idx = xf[:, ju] * total + xi[:, iu]
    vj_idx = xf[:, iu] * total + xi[:, ju]
    idx = jnp.concatenate([vi_idx, vj_idx], axis=1)
    idx = jnp.pad(idx, ((0, 0), (0, NIDX_PAD - NIDX)))
    idx_main = idx.reshape(batch, 6, 128)
    val = jnp.broadcast_to(x_val.astype(jnp.float32)[:, :, None],
                           (batch, NF, D))

    mesh = plsc.VectorSubcoreMesh(core_axis_name="c", subcore_axis_name="s",
                                  num_cores=NC, num_subcores=NS)
    cp = pltpu.CompilerParams()
    if "needs_layout_passes" in pltpu.CompilerParams.__dataclass_fields__:
        cp = dataclasses.replace(cp, needs_layout_passes=False)
    if "use_tc_tiling_on_sc" in pltpu.CompilerParams.__dataclass_fields__:
        cp = dataclasses.replace(cp, use_tc_tiling_on_sc=False)
    k = pl.kernel(
        functools.partial(_ffm_body, rows_per_w),
        out_type=jax.ShapeDtypeStruct((batch,), jnp.float32),
        mesh=mesh,
        scratch_types=[
            pltpu.VMEM((6, 128), jnp.int32),
            pltpu.VMEM((NF, D), jnp.float32),
            pltpu.VMEM((NIDX_PAD, D), jnp.float32),
            pltpu.VMEM((D,), jnp.float32),
            pltpu.VMEM((rows_per_w,), jnp.float32),
            pltpu.SemaphoreType.DMA,
        ],
        compiler_params=cp,
    )
    return k(emb_flat, idx_main, val)


# speedup vs baseline: 1.0015x; 1.0015x over previous
import dataclasses
import functools

import numpy as np
import jax
import jax.numpy as jnp
from jax import lax
from jax.experimental import pallas as pl
from jax.experimental.pallas import tpu as pltpu
from jax.experimental.pallas import tpu_sc as plsc

NF = 26
D = 16
P = NF * (NF - 1) // 2
NIDX = 2 * P
NIDX_PAD = 768
NCHUNK = 6
GRP = 16
NC, NS = 2, 16
NW = NC * NS

_IU, _JU = np.triu_indices(NF, k=1)
_IU_OH = np.zeros((NF, P), np.float32)
_JU_OH = np.zeros((NF, P), np.float32)
_IU_OH[_IU, np.arange(P)] = 1.0
_JU_OH[_JU, np.arange(P)] = 1.0


def _compute_row(rows_v, val_v):
    vbc = [val_v[i] for i in range(NF)]
    acc = jnp.zeros((D,), jnp.float32)
    p = 0
    for i in range(NF - 1):
        acc_i = jnp.zeros((D,), jnp.float32)
        for j in range(i + 1, NF):
            acc_i = acc_i + rows_v[p] * rows_v[P + p] * vbc[j]
            p += 1
        acc = acc + acc_i * vbc[i]
    return acc


def _issue_gathers(emb_hbm, idx_v, rows_v, sem):
    copies = []
    for c in range(NCHUNK):
        copies.append(pltpu.async_copy(
            emb_hbm.at[idx_v.at[c]],
            rows_v.at[pl.ds(c * 128, 128)], sem))
    return copies


def _ffm_body(rows_per_w, emb_hbm, idx_hbm, val_hbm, out_hbm,
              idx_v, val_v, rows_a, rows_b, acc_v, out_v, sem_a, sem_b):
    core = lax.axis_index("c")
    sub = lax.axis_index("s")
    wid = sub * NC + core
    base = wid * rows_per_w
    acc_v[...] = jnp.zeros((D,), jnp.float32)

    def stage_group(r):
        g = pl.multiple_of(base + r, GRP)
        pltpu.sync_copy(idx_hbm.at[pl.ds(g, GRP)], idx_v)
        pltpu.sync_copy(val_hbm.at[pl.ds(g, GRP)], val_v)

    def finish_row(r, acc):
        s = jnp.sum(acc)
        lane = lax.rem(r, D)
        sel = lax.iota(jnp.int32, D) == lane
        acc_v[...] = acc_v[...] + jnp.where(sel, s, jnp.float32(0.0))

        @pl.when(lane == D - 1)
        def _flush():
            off = pl.multiple_of((r // D) * D, D)
            out_v[pl.ds(off, D)] = acc_v[...]
            acc_v[...] = jnp.zeros((D,), jnp.float32)

    @pl.loop(0, rows_per_w)
    def _rows(r):
        row = base + r
        pltpu.sync_copy(idx_hbm.at[row], idx_v)
        pltpu.sync_copy(val_hbm.at[row], val_v)
        cps_a = _issue_gathers(emb_hbm, idx_v, rows_a, sem_a)
        for cp in cps_a:
            cp.wait()
        acc0 = _compute_row(rows_a, val_v)
        finish_row(r, acc0)

    pltpu.sync_copy(out_v, out_hbm.at[pl.ds(base, rows_per_w)])


def kernel(x_field, x, x_val, emb):
    batch = x.shape[0]
    total = emb.shape[1]
    rows_per_w = batch // NW

    emb_flat = emb.reshape(NF * total, D)
    xf_f = x_field.astype(jnp.float32)
    xi_f = x.astype(jnp.float32)
    iu_oh = jnp.asarray(_IU_OH)
    ju_oh = jnp.asarray(_JU_OH)
    mm = functools.partial(jnp.matmul, precision=lax.Precision.HIGHEST)
    vi_idx = mm(xf_f, ju_oh) * total + mm(xi_f, iu_oh)
    vj_idx = mm(xf_f, iu_oh) * total + mm(xi_f, ju_oh)
    idx = jnp.concatenate([vi_idx, vj_idx], axis=1).astype(jnp.int32)
    idx = jnp.pad(idx, ((0, 0), (0, NIDX_PAD - NIDX)))
    idx_main = idx.reshape(batch, NCHUNK, 128)
    val = jnp.broadcast_to(x_val.astype(jnp.float32)[:, :, None],
                           (batch, NF, D))

    mesh = plsc.VectorSubcoreMesh(core_axis_name="c", subcore_axis_name="s",
                                  num_cores=NC, num_subcores=NS)
    cp = pltpu.CompilerParams()
    if "needs_layout_passes" in pltpu.CompilerParams.__dataclass_fields__:
        cp = dataclasses.replace(cp, needs_layout_passes=False)
    if "use_tc_tiling_on_sc" in pltpu.CompilerParams.__dataclass_fields__:
        cp = dataclasses.replace(cp, use_tc_tiling_on_sc=False)
    k = pl.kernel(
        functools.partial(_ffm_body, rows_per_w),
        out_type=jax.ShapeDtypeStruct((batch,), jnp.float32),
        mesh=mesh,
        scratch_types=[
            pltpu.VMEM((NCHUNK, 128), jnp.int32),
            pltpu.VMEM((NF, D), jnp.float32),
            pltpu.VMEM((NIDX_PAD, D), jnp.float32),
            pltpu.VMEM((NIDX_PAD, D), jnp.float32),
            pltpu.VMEM((D,), jnp.float32),
            pltpu.VMEM((rows_per_w,), jnp.float32),
            pltpu.SemaphoreType.DMA,
            pltpu.SemaphoreType.DMA,
        ],
        compiler_params=cp,
    )
    return k(emb_flat, idx_main, val)


# speedup vs baseline: 1.0029x; 1.0013x over previous
import dataclasses
import functools

import numpy as np
import jax
import jax.numpy as jnp
from jax import lax
from jax.experimental import pallas as pl
from jax.experimental.pallas import tpu as pltpu
from jax.experimental.pallas import tpu_sc as plsc

NF = 26
D = 16
P = NF * (NF - 1) // 2
NIDX = 2 * P
NIDX_PAD = 768
NCHUNK = 6
GRP = 16
NC, NS = 2, 16
NW = NC * NS

_IU, _JU = np.triu_indices(NF, k=1)
_IU_OH = np.zeros((NF, P), np.float32)
_JU_OH = np.zeros((NF, P), np.float32)
_IU_OH[_IU, np.arange(P)] = 1.0
_JU_OH[_JU, np.arange(P)] = 1.0


def _compute_row(rows_v, val_v, vo):
    vbc = [val_v[vo + i] for i in range(NF)]
    acc = jnp.zeros((D,), jnp.float32)
    p = 0
    for i in range(NF - 1):
        acc_i = jnp.zeros((D,), jnp.float32)
        for j in range(i + 1, NF):
            acc_i = acc_i + rows_v[p] * rows_v[P + p] * vbc[j]
            p += 1
        acc = acc + acc_i * vbc[i]
    return acc


def _issue_gathers(emb_hbm, idx_v, io, rows_v, sem):
    copies = []
    for c in range(NCHUNK):
        copies.append(pltpu.async_copy(
            emb_hbm.at[idx_v.at[io + c]],
            rows_v.at[pl.ds(c * 128, 128)], sem))
    return copies


def _ffm_body(rows_per_w, emb_hbm, idx_hbm, val_hbm, out_hbm,
              idx_v, val_v, rows_a, rows_b, acc_v, out_v, sem_a, sem_b):
    core = lax.axis_index("c")
    sub = lax.axis_index("s")
    wid = sub * NC + core
    base = wid * rows_per_w
    acc_v[...] = jnp.zeros((D,), jnp.float32)

    def stage_group(r):
        g = pl.multiple_of(base + r, GRP)
        pltpu.sync_copy(idx_hbm.at[pl.ds(g * NCHUNK, GRP * NCHUNK)], idx_v)
        pltpu.sync_copy(val_hbm.at[pl.ds(g * NF, GRP * NF)], val_v)

    def finish_row(r, acc):
        s = jnp.sum(acc)
        lane = lax.rem(r, D)
        sel = lax.iota(jnp.int32, D) == lane
        acc_v[...] = acc_v[...] + jnp.where(sel, s, jnp.float32(0.0))

        @pl.when(lane == D - 1)
        def _flush():
            off = pl.multiple_of((r // D) * D, D)
            out_v[pl.ds(off, D)] = acc_v[...]
            acc_v[...] = jnp.zeros((D,), jnp.float32)

    @pl.loop(0, rows_per_w, step=2)
    def _rows(r):
        rmod = lax.rem(r, GRP)

        @pl.when(rmod == 0)
        def _stage():
            stage_group(r)

        io = rmod * NCHUNK
        vo = rmod * NF
        cps_a = _issue_gathers(emb_hbm, idx_v, io, rows_a, sem_a)
        cps_b = _issue_gathers(emb_hbm, idx_v, io + NCHUNK, rows_b, sem_b)
        for cp in cps_a:
            cp.wait()
        acc0 = _compute_row(rows_a, val_v, vo)
        for cp in cps_b:
            cp.wait()
        finish_row(r, acc0)
        acc1 = _compute_row(rows_b, val_v, vo + NF)
        finish_row(r + 1, acc1)

    pltpu.sync_copy(out_v, out_hbm.at[pl.ds(base, rows_per_w)])


def kernel(x_field, x, x_val, emb):
    batch = x.shape[0]
    total = emb.shape[1]
    rows_per_w = batch // NW

    emb_flat = emb.reshape(NF * total, D)
    xf_f = x_field.astype(jnp.float32)
    xi_f = x.astype(jnp.float32)
    iu_oh = jnp.asarray(_IU_OH)
    ju_oh = jnp.asarray(_JU_OH)
    mm = functools.partial(jnp.matmul, precision=lax.Precision.HIGHEST)
    vi_idx = mm(xf_f, ju_oh) * total + mm(xi_f, iu_oh)
    vj_idx = mm(xf_f, iu_oh) * total + mm(xi_f, ju_oh)
    idx = jnp.concatenate([vi_idx, vj_idx], axis=1).astype(jnp.int32)
    idx = jnp.pad(idx, ((0, 0), (0, NIDX_PAD - NIDX)))
    idx_main = idx.reshape(batch * NCHUNK, 128)
    val = jnp.broadcast_to(x_val.astype(jnp.float32)[:, :, None],
                           (batch, NF, D)).reshape(batch * NF, D)

    mesh = plsc.VectorSubcoreMesh(core_axis_name="c", subcore_axis_name="s",
                                  num_cores=NC, num_subcores=NS)
    cp = pltpu.CompilerParams()
    if "needs_layout_passes" in pltpu.CompilerParams.__dataclass_fields__:
        cp = dataclasses.replace(cp, needs_layout_passes=False)
    if "use_tc_tiling_on_sc" in pltpu.CompilerParams.__dataclass_fields__:
        cp = dataclasses.replace(cp, use_tc_tiling_on_sc=False)
    k = pl.kernel(
        functools.partial(_ffm_body, rows_per_w),
        out_type=jax.ShapeDtypeStruct((batch,), jnp.float32),
        mesh=mesh,
        scratch_types=[
            pltpu.VMEM((GRP * NCHUNK, 128), jnp.int32),
            pltpu.VMEM((GRP * NF, D), jnp.float32),
            pltpu.VMEM((NIDX_PAD, D), jnp.float32),
            pltpu.VMEM((NIDX_PAD, D), jnp.float32),
            pltpu.VMEM((D,), jnp.float32),
            pltpu.VMEM((rows_per_w,), jnp.float32),
            pltpu.SemaphoreType.DMA,
            pltpu.SemaphoreType.DMA,
        ],
        compiler_params=cp,
    )
    return k(emb_flat, idx_main, val)


# speedup vs baseline: 4.2771x; 4.2649x over previous
import dataclasses
import functools

import numpy as np
import jax
import jax.numpy as jnp
from jax import lax
from jax.experimental import pallas as pl
from jax.experimental.pallas import tpu as pltpu
from jax.experimental.pallas import tpu_sc as plsc

NF = 26
D = 16
P = NF * (NF - 1) // 2
ROWW = NF * D
GRP = 16
NC, NS = 2, 16
NW = NC * NS

_IU, _JU = np.triu_indices(NF, k=1)


def _compute_row(e_v, xfcol, val_v, vo, ebase):
    vbc = [val_v[vo + i] for i in range(NF)]
    acc = jnp.zeros((D,), jnp.float32)
    p = 0
    for i in range(NF - 1):
        acc_i = jnp.zeros((D,), jnp.float32)
        for j in range(i + 1, NF):
            vi = e_v[ebase + i, pl.ds(xfcol[j], D)]
            vj = e_v[ebase + j, pl.ds(xfcol[i], D)]
            acc_i = acc_i + vi * vj * vbc[j]
            p += 1
        acc = acc + acc_i * vbc[i]
    return acc


def _ffm_body(rows_per_w, emb_hbm, xpair_hbm, xf_hbm, val_hbm, out_hbm,
              xpair_v, xf_v, val_v, e_v, acc_v, out_v, sem):
    core = lax.axis_index("c")
    sub = lax.axis_index("s")
    wid = sub * NC + core
    base = wid * rows_per_w
    acc_v[...] = jnp.zeros((D,), jnp.float32)

    def finish_row(r, acc):
        s = jnp.sum(acc)
        lane = lax.rem(r, D)
        sel = lax.iota(jnp.int32, D) == lane
        acc_v[...] = acc_v[...] + jnp.where(sel, s, jnp.float32(0.0))

        @pl.when(lane == D - 1)
        def _flush():
            off = pl.multiple_of((r // D) * D, D)
            out_v[pl.ds(off, D)] = acc_v[...]
            acc_v[...] = jnp.zeros((D,), jnp.float32)

    @pl.loop(0, rows_per_w, step=2)
    def _rows(r):
        rmod = lax.rem(r, GRP)

        @pl.when(rmod == 0)
        def _stage():
            g = pl.multiple_of(base + r, GRP)
            pltpu.sync_copy(xpair_hbm.at[pl.ds(g // 2, GRP // 2)], xpair_v)
            pltpu.sync_copy(xf_hbm.at[pl.ds(g * 32, GRP * 32)], xf_v)
            pltpu.sync_copy(val_hbm.at[pl.ds(g * NF, GRP * NF)], val_v)

        g2 = rmod // 2
        pltpu.async_copy(emb_hbm.at[xpair_v.at[g2]], e_v, sem).wait()

        vo = rmod * NF

        def xfcols(row_in_grp):
            o = row_in_grp * 32
            a = xf_v[pl.ds(o, D)]
            b = xf_v[pl.ds(o + D, D)]
            return [(a[j] if j < D else b[j - D]) * D for j in range(NF)]

        xfcol0 = xfcols(rmod)
        xfcol1 = xfcols(rmod + 1)
        acc0 = _compute_row(e_v, xfcol0, val_v, vo, 0)
        finish_row(r, acc0)
        acc1 = _compute_row(e_v, xfcol1, val_v, vo + NF, NF)
        finish_row(r + 1, acc1)

    pltpu.sync_copy(out_v, out_hbm.at[pl.ds(base, rows_per_w)])


def kernel(x_field, x, x_val, emb):
    batch = x.shape[0]
    total = emb.shape[1]
    rows_per_w = batch // NW

    emb_t = jnp.transpose(emb, (1, 0, 2)).reshape(total, ROWW)
    xpair = x.astype(jnp.int32).reshape(batch // 2, 2 * NF)
    xf_flat = jnp.pad(x_field.astype(jnp.int32),
                      ((0, 0), (0, 32 - NF))).reshape(batch * 32)
    val = jnp.broadcast_to(x_val.astype(jnp.float32)[:, :, None],
                           (batch, NF, D)).reshape(batch * NF, D)

    mesh = plsc.VectorSubcoreMesh(core_axis_name="c", subcore_axis_name="s",
                                  num_cores=NC, num_subcores=NS)
    cp = pltpu.CompilerParams()
    if "needs_layout_passes" in pltpu.CompilerParams.__dataclass_fields__:
        cp = dataclasses.replace(cp, needs_layout_passes=False)
    if "use_tc_tiling_on_sc" in pltpu.CompilerParams.__dataclass_fields__:
        cp = dataclasses.replace(cp, use_tc_tiling_on_sc=False)
    k = pl.kernel(
        functools.partial(_ffm_body, rows_per_w),
        out_type=jax.ShapeDtypeStruct((batch,), jnp.float32),
        mesh=mesh,
        scratch_types=[
            pltpu.VMEM((GRP // 2, 2 * NF), jnp.int32),
            pltpu.VMEM((GRP * 32,), jnp.int32),
            pltpu.VMEM((GRP * NF, D), jnp.float32),
            pltpu.VMEM((2 * NF, ROWW), jnp.float32),
            pltpu.VMEM((D,), jnp.float32),
            pltpu.VMEM((rows_per_w,), jnp.float32),
            pltpu.SemaphoreType.DMA,
        ],
        compiler_params=cp,
    )
    return k(emb_t, xpair, xf_flat, val)


# speedup vs baseline: 4.9115x; 1.1483x over previous
import dataclasses
import functools

import numpy as np
import jax
import jax.numpy as jnp
from jax import lax
from jax.experimental import pallas as pl
from jax.experimental.pallas import tpu as pltpu
from jax.experimental.pallas import tpu_sc as plsc

NF = 26
D = 16
P = NF * (NF - 1) // 2
ROWW = NF * D
GRP = 16
NC, NS = 2, 16
NW = NC * NS

_IU, _JU = np.triu_indices(NF, k=1)


def _compute_row(e_v, xfcol, vbc, ebase):
    acc = jnp.zeros((D,), jnp.float32)
    p = 0
    for i in range(NF - 1):
        acc_i = jnp.zeros((D,), jnp.float32)
        for j in range(i + 1, NF):
            vi = e_v[ebase + i, pl.ds(xfcol[j], D)]
            vj = e_v[ebase + j, pl.ds(xfcol[i], D)]
            acc_i = acc_i + vi * vj * vbc[j]
            p += 1
        acc = acc + acc_i * vbc[i]
    return acc


def _ffm_body(rows_per_w, emb_hbm, xpair_hbm, xf_hbm, val_hbm, out_hbm,
              xpair_v, xf_v, val_v, e_a, e_b, acc_v, out_v, sem_a, sem_b):
    core = lax.axis_index("c")
    sub = lax.axis_index("s")
    wid = sub * NC + core
    base = wid * rows_per_w
    npairs = rows_per_w // 2
    acc_v[...] = jnp.zeros((D,), jnp.float32)

    pltpu.sync_copy(xpair_hbm.at[pl.ds(base // 2, npairs)], xpair_v)
    pltpu.sync_copy(xf_hbm.at[pl.ds(base * 32, rows_per_w * 32)], xf_v)
    pltpu.sync_copy(val_hbm.at[pl.ds(base * 32, rows_per_w * 32)], val_v)

    def finish_row(r, acc):
        s = jnp.sum(acc)
        lane = lax.rem(r, D)
        sel = lax.iota(jnp.int32, D) == lane
        acc_v[...] = acc_v[...] + jnp.where(sel, s, jnp.float32(0.0))

        @pl.when(lane == D - 1)
        def _flush():
            off = pl.multiple_of((r // D) * D, D)
            out_v[pl.ds(off, D)] = acc_v[...]
            acc_v[...] = jnp.zeros((D,), jnp.float32)

    def xfcols(r):
        o = r * 32
        a = xf_v[pl.ds(o, D)]
        b = xf_v[pl.ds(o + D, D)]
        return [(a[j] if j < D else b[j - D]) * D for j in range(NF)]

    def vbcs(r):
        o = r * 32
        a = val_v[pl.ds(o, D)]
        b = val_v[pl.ds(o + D, D)]
        return [jnp.full((D,), a[j] if j < D else b[j - D], jnp.float32)
                for j in range(NF)]

    def compute_pair(e_v, r):
        acc0 = _compute_row(e_v, xfcols(r), vbcs(r), 0)
        finish_row(r, acc0)
        acc1 = _compute_row(e_v, xfcols(r + 1), vbcs(r + 1), NF)
        finish_row(r + 1, acc1)

    def issue(pair_idx, e_v, sem):
        p = jnp.minimum(pair_idx, npairs - 1)
        return pltpu.async_copy(emb_hbm.at[xpair_v.at[p]], e_v, sem)

    issue(0, e_a, sem_a)

    @pl.loop(0, rows_per_w, step=4)
    def _rows(r):
        q = r // 2
        issue(q + 1, e_b, sem_b)
        pltpu.make_async_copy(emb_hbm.at[xpair_v.at[0]], e_a, sem_a).wait()
        compute_pair(e_a, r)
        issue(q + 2, e_a, sem_a)
        pltpu.make_async_copy(emb_hbm.at[xpair_v.at[0]], e_b, sem_b).wait()
        compute_pair(e_b, r + 2)

    pltpu.make_async_copy(emb_hbm.at[xpair_v.at[0]], e_a, sem_a).wait()
    pltpu.sync_copy(out_v, out_hbm.at[pl.ds(base, rows_per_w)])


def kernel(x_field, x, x_val, emb):
    batch = x.shape[0]
    total = emb.shape[1]
    rows_per_w = batch // NW

    emb_t = jnp.transpose(emb, (1, 0, 2)).reshape(total, ROWW)
    xpair = x.astype(jnp.int32).reshape(batch // 2, 2 * NF)
    xf_flat = jnp.pad(x_field.astype(jnp.int32),
                      ((0, 0), (0, 32 - NF))).reshape(batch * 32)
    val = jnp.pad(x_val.astype(jnp.float32),
                  ((0, 0), (0, 32 - NF))).reshape(batch * 32)

    mesh = plsc.VectorSubcoreMesh(core_axis_name="c", subcore_axis_name="s",
                                  num_cores=NC, num_subcores=NS)
    cp = pltpu.CompilerParams()
    if "needs_layout_passes" in pltpu.CompilerParams.__dataclass_fields__:
        cp = dataclasses.replace(cp, needs_layout_passes=False)
    if "use_tc_tiling_on_sc" in pltpu.CompilerParams.__dataclass_fields__:
        cp = dataclasses.replace(cp, use_tc_tiling_on_sc=False)
    k = pl.kernel(
        functools.partial(_ffm_body, rows_per_w),
        out_type=jax.ShapeDtypeStruct((batch,), jnp.float32),
        mesh=mesh,
        scratch_types=[
            pltpu.VMEM((rows_per_w // 2, 2 * NF), jnp.int32),
            pltpu.VMEM((rows_per_w * 32,), jnp.int32),
            pltpu.VMEM((rows_per_w * 32,), jnp.float32),
            pltpu.VMEM((2 * NF, ROWW), jnp.float32),
            pltpu.VMEM((2 * NF, ROWW), jnp.float32),
            pltpu.VMEM((D,), jnp.float32),
            pltpu.VMEM((rows_per_w,), jnp.float32),
            pltpu.SemaphoreType.DMA,
            pltpu.SemaphoreType.DMA,
        ],
        compiler_params=cp,
    )
    return k(emb_t, xpair, xf_flat, val)


# speedup vs baseline: 4.9338x; 1.0045x over previous
import dataclasses
import functools

import numpy as np
import jax
import jax.numpy as jnp
from jax import lax
from jax.experimental import pallas as pl
from jax.experimental.pallas import tpu as pltpu
from jax.experimental.pallas import tpu_sc as plsc

NF = 26
D = 16
P = NF * (NF - 1) // 2
ROWW = NF * D
GRP = 16
NC, NS = 2, 16
NW = NC * NS

_IU, _JU = np.triu_indices(NF, k=1)


def _compute_row(e_v, xfcol, vbc, ebase):
    acc = jnp.zeros((D,), jnp.float32)
    p = 0
    for i in range(NF - 1):
        acc_i = jnp.zeros((D,), jnp.float32)
        for j in range(i + 1, NF):
            vi = e_v[ebase + i, pl.ds(xfcol[j], D)]
            vj = e_v[ebase + j, pl.ds(xfcol[i], D)]
            acc_i = acc_i + vi * vj * vbc[j]
            p += 1
        acc = acc + acc_i * vbc[i]
    return acc


def _ffm_body(rows_per_w, emb_hbm, xpair_hbm, xf_hbm, val_hbm, out_hbm,
              xpair_v, xf_v, val_v, e_a, e_b, acc_v, out_v, sem_a, sem_b):
    core = lax.axis_index("c")
    sub = lax.axis_index("s")
    wid = sub * NC + core
    base = wid * rows_per_w
    npairs = rows_per_w // 2
    acc_v[...] = jnp.zeros((D,), jnp.float32)

    pltpu.sync_copy(xpair_hbm.at[pl.ds(base, rows_per_w)], xpair_v)
    pltpu.sync_copy(xf_hbm.at[pl.ds(base * 32, rows_per_w * 32)], xf_v)
    pltpu.sync_copy(val_hbm.at[pl.ds(base * 32, rows_per_w * 32)], val_v)

    def finish_row(r, acc):
        s = jnp.sum(acc)
        lane = lax.rem(r, D)
        sel = lax.iota(jnp.int32, D) == lane
        acc_v[...] = acc_v[...] + jnp.where(sel, s, jnp.float32(0.0))

        @pl.when(lane == D - 1)
        def _flush():
            off = pl.multiple_of((r // D) * D, D)
            out_v[pl.ds(off, D)] = acc_v[...]
            acc_v[...] = jnp.zeros((D,), jnp.float32)

    def xfcols(r):
        o = r * 32
        a = xf_v[pl.ds(o, D)]
        b = xf_v[pl.ds(o + D, D)]
        return [(a[j] if j < D else b[j - D]) * D for j in range(NF)]

    def vbcs(r):
        o = r * 32
        a = val_v[pl.ds(o, D)]
        b = val_v[pl.ds(o + D, D)]
        return [jnp.full((D,), a[j] if j < D else b[j - D], jnp.float32)
                for j in range(NF)]

    def compute_pair(e_v, r):
        acc0 = _compute_row(e_v, xfcols(r), vbcs(r), 0)
        finish_row(r, acc0)
        acc1 = _compute_row(e_v, xfcols(r + 1), vbcs(r + 1), NF)
        finish_row(r + 1, acc1)

    def issue(pair_idx, e_v, sem):
        p = jnp.minimum(pair_idx, npairs - 1)
        pltpu.async_copy(emb_hbm.at[xpair_v.at[2 * p]],
                         e_v.at[pl.ds(0, NF)], sem)
        pltpu.async_copy(emb_hbm.at[xpair_v.at[2 * p + 1]],
                         e_v.at[pl.ds(NF, NF)], sem)

    def drain(e_v, sem):
        pltpu.make_async_copy(emb_hbm.at[xpair_v.at[0]],
                              e_v.at[pl.ds(0, NF)], sem).wait()
        pltpu.make_async_copy(emb_hbm.at[xpair_v.at[0]],
                              e_v.at[pl.ds(NF, NF)], sem).wait()

    issue(0, e_a, sem_a)

    @pl.loop(0, rows_per_w, step=4)
    def _rows(r):
        q = r // 2
        issue(q + 1, e_b, sem_b)
        drain(e_a, sem_a)
        compute_pair(e_a, r)
        issue(q + 2, e_a, sem_a)
        drain(e_b, sem_b)
        compute_pair(e_b, r + 2)

    drain(e_a, sem_a)
    pltpu.sync_copy(out_v, out_hbm.at[pl.ds(base, rows_per_w)])


def kernel(x_field, x, x_val, emb):
    batch = x.shape[0]
    total = emb.shape[1]
    rows_per_w = batch // NW

    emb_t = jnp.transpose(emb, (1, 0, 2)).reshape(total, ROWW)
    xpair = x.astype(jnp.int32)
    xf_flat = jnp.pad(x_field.astype(jnp.int32),
                      ((0, 0), (0, 32 - NF))).reshape(batch * 32)
    val = jnp.pad(x_val.astype(jnp.float32),
                  ((0, 0), (0, 32 - NF))).reshape(batch * 32)

    mesh = plsc.VectorSubcoreMesh(core_axis_name="c", subcore_axis_name="s",
                                  num_cores=NC, num_subcores=NS)
    cp = pltpu.CompilerParams()
    if "needs_layout_passes" in pltpu.CompilerParams.__dataclass_fields__:
        cp = dataclasses.replace(cp, needs_layout_passes=False)
    if "use_tc_tiling_on_sc" in pltpu.CompilerParams.__dataclass_fields__:
        cp = dataclasses.replace(cp, use_tc_tiling_on_sc=False)
    k = pl.kernel(
        functools.partial(_ffm_body, rows_per_w),
        out_type=jax.ShapeDtypeStruct((batch,), jnp.float32),
        mesh=mesh,
        scratch_types=[
            pltpu.VMEM((rows_per_w, NF), jnp.int32),
            pltpu.VMEM((rows_per_w * 32,), jnp.int32),
            pltpu.VMEM((rows_per_w * 32,), jnp.float32),
            pltpu.VMEM((2 * NF, ROWW), jnp.float32),
            pltpu.VMEM((2 * NF, ROWW), jnp.float32),
            pltpu.VMEM((D,), jnp.float32),
            pltpu.VMEM((rows_per_w,), jnp.float32),
            pltpu.SemaphoreType.DMA,
            pltpu.SemaphoreType.DMA,
        ],
        compiler_params=cp,
    )
    return k(emb_t, xpair, xf_flat, val)
